# baseline (speedup 1.0000x reference)
import functools

import jax
import jax.numpy as jnp
from jax import lax
from jax.experimental import pallas as pl
from jax.experimental.pallas import tpu as pltpu
from jax.experimental.pallas import tpu_sc as plsc

N = 10000
E = 320000
D = 64
D_IN = 128
CORES = 2
SUBCORES = 16
TILES = CORES * SUBCORES
CHUNK = 128
DH = D // CORES
CH_PER_TILE = 158
EPAD = SUBCORES * CH_PER_TILE * CHUNK
NPAD = 10240
ZROWS = NPAD // SUBCORES
OROWS = N // SUBCORES

_mesh = plsc.VectorSubcoreMesh(core_axis_name="c", subcore_axis_name="s")
_f32 = jnp.float32


def _seg_body(g_hbm, row_hbm, col_hbm, zero_hbm, out_hbm, row_v, col_v,
              rows0, rows1, gs0, gs1, acc):
    c = lax.axis_index("c")
    s = lax.axis_index("s")
    pltpu.sync_copy(zero_hbm, acc.at[pl.ds(s * ZROWS, ZROWS)])
    pltpu.sync_copy(row_hbm.at[c].at[s], row_v)
    pltpu.sync_copy(col_hbm.at[s], col_v)
    plsc.subcore_barrier()

    pltpu.async_copy(g_hbm.at[row_v.at[0]], rows0, gs0)

    @pl.loop(0, CH_PER_TILE, step=2)
    def _(j):
        pltpu.async_copy(g_hbm.at[row_v.at[j + 1]], rows1, gs1)
        pltpu.make_async_copy(g_hbm.at[row_v.at[j]], rows0, gs0).wait()
        pltpu.sync_copy(rows0, acc.at[col_v.at[j]], add=True)

        @pl.when(j + 2 < CH_PER_TILE)
        def _():
            pltpu.async_copy(g_hbm.at[row_v.at[j + 2]], rows0, gs0)

        pltpu.make_async_copy(g_hbm.at[row_v.at[j + 1]], rows1, gs1).wait()
        pltpu.sync_copy(rows1, acc.at[col_v.at[j + 1]], add=True)

    plsc.subcore_barrier()
    pltpu.sync_copy(acc.at[pl.ds(s * ZROWS, ZROWS)],
                    out_hbm.at[pl.ds(s * ZROWS, ZROWS), c])


@jax.jit
def _seg_sum(g2, row4, col3, zeros_slab):
    return pl.kernel(
        _seg_body,
        out_type=jax.ShapeDtypeStruct((NPAD, CORES, DH), _f32),
        mesh=_mesh,
        scratch_types=[
            pltpu.VMEM((CH_PER_TILE, CHUNK), jnp.int32),
            pltpu.VMEM((CH_PER_TILE, CHUNK), jnp.int32),
            pltpu.VMEM((CHUNK, DH), _f32),
            pltpu.VMEM((CHUNK, DH), _f32),
            pltpu.SemaphoreType.DMA,
            pltpu.SemaphoreType.DMA,
            pltpu.VMEM_SHARED((NPAD, DH), _f32),
        ],
        compiler_params=pltpu.CompilerParams(use_tc_tiling_on_sc=False),
    )(g2, row4, col3, zeros_slab)


def _deg_body(col_hbm, zero_hbm, out_hbm, col_v, acc):
    c = lax.axis_index("c")
    s = lax.axis_index("s")
    w = c * SUBCORES + s
    pltpu.sync_copy(zero_hbm, acc)
    pltpu.sync_copy(col_hbm.at[s], col_v)
    ones = jnp.full((16,), 1.0, _f32)
    half = CH_PER_TILE // CORES

    @pl.loop(0, half)
    def _(j):
        @pl.loop(0, CHUNK // 16)
        def _(i):
            idx = col_v[c * half + j, pl.ds(i * 16, 16)]
            plsc.addupdate_scatter(acc, [idx], ones)

    pltpu.sync_copy(acc, out_hbm.at[w])


@jax.jit
def _deg(col3, zeros_n):
    return pl.kernel(
        _deg_body,
        out_type=jax.ShapeDtypeStruct((TILES, NPAD), _f32),
        mesh=_mesh,
        scratch_types=[
            pltpu.VMEM((CH_PER_TILE, CHUNK), jnp.int32),
            pltpu.VMEM((NPAD,), _f32),
        ],
        compiler_params=pltpu.CompilerParams(needs_layout_passes=False),
    )(col3, zeros_n)


_DOT = functools.partial(jnp.dot, preferred_element_type=_f32,
                         precision=lax.Precision.HIGHEST)


def _pre_a_body(x_ref, gam_ref, bet_ref, W1_ref, b1_ref, tw0_ref,
                x0_ref, h_ref, out0_ref):
    x = x_ref[...]
    mean = jnp.mean(x, axis=0, keepdims=True)
    xc = x - mean
    var = jnp.mean(xc * xc, axis=0, keepdims=True)
    x0 = xc * lax.rsqrt(var + 1e-5)
    x0 = x0 * gam_ref[...][None, :] + bet_ref[...][None, :]
    x0_ref[...] = x0
    h = jnp.maximum(_DOT(x0, W1_ref[...]) + b1_ref[...][None, :], 0.0)
    h_ref[...] = h
    out0_ref[...] = _DOT(h, tw0_ref[...])


@jax.jit
def _pre_a(x, gam, bet, W1, b1, tw0):
    return pl.pallas_call(
        _pre_a_body,
        out_shape=(
            jax.ShapeDtypeStruct((N, D_IN), _f32),
            jax.ShapeDtypeStruct((N, D), _f32),
            jax.ShapeDtypeStruct((N, D), _f32),
        ),
    )(x, gam, bet, W1, b1, tw0)


def _pre_b_body(h_ref, degp_ref, g0_ref, dv_ref):
    deg = lax.dot_general(degp_ref[...], jnp.ones((TILES, 1), _f32),
                          (((0,), (0,)), ((), ())),
                          preferred_element_type=_f32,
                          precision=lax.Precision.HIGHEST)[:N]
    dinv = jnp.where(deg > 0.0, lax.rsqrt(jnp.maximum(deg, 1e-30)), 0.0)
    dv = jnp.broadcast_to(dinv, (N, D))
    dv_ref[...] = dv
    g0_ref[...] = h_ref[...] * dv


@jax.jit
def _pre_b(h, degp):
    return pl.pallas_call(
        _pre_b_body,
        out_shape=(
            jax.ShapeDtypeStruct((N, D), _f32),
            jax.ShapeDtypeStruct((N, D), _f32),
        ),
    )(h, degp)


def _hop_body(p_ref, dv_ref, acc_ref, tw_ref, out_ref, g_ref):
    dv = dv_ref[...]
    h = p_ref[...][:N] * dv
    out_ref[...] = acc_ref[...] + _DOT(h, tw_ref[...])
    g_ref[...] = h * dv


@jax.jit
def _hop(p, dv, acc, tw):
    return pl.pallas_call(
        _hop_body,
        out_shape=(
            jax.ShapeDtypeStruct((N, D), _f32),
            jax.ShapeDtypeStruct((N, D), _f32),
        ),
    )(p, dv, acc, tw)


def _mid_body(o_ref, bt_ref, W2_ref, b2_ref, tw0_ref, dv_ref,
              out0_ref, g_ref):
    h1 = jnp.maximum(o_ref[...] + bt_ref[...][None, :], 0.0)
    h2 = jnp.maximum(_DOT(h1, W2_ref[...]) + b2_ref[...][None, :], 0.0)
    out0_ref[...] = _DOT(h2, tw0_ref[...])
    g_ref[...] = h2 * dv_ref[...]


@jax.jit
def _mid(o, bt, W2, b2, tw0, dv):
    return pl.pallas_call(
        _mid_body,
        out_shape=(
            jax.ShapeDtypeStruct((N, D), _f32),
            jax.ShapeDtypeStruct((N, D), _f32),
        ),
    )(o, bt, W2, b2, tw0, dv)


def _fin_body(o_ref, bt_ref, x0_ref, W3_ref, b3_ref, W4_ref, b4_ref, y_ref):
    h = jnp.maximum(o_ref[...] + bt_ref[...][None, :], 0.0)
    W3 = W3_ref[...]
    a = _DOT(x0_ref[...], W3[:D_IN]) + _DOT(h, W3[D_IN:]) + b3_ref[...][None, :]
    a = jnp.maximum(a, 0.0)
    z = _DOT(a, W4_ref[...]) + b4_ref[...][None, :]
    z = z - jnp.max(z, axis=-1, keepdims=True)
    ez = jnp.exp(z)
    y_ref[...] = ez / jnp.sum(ez, axis=-1, keepdims=True)


@jax.jit
def _fin(o, bt, x0, W3, b3, W4, b4):
    return pl.pallas_call(
        _fin_body,
        out_shape=jax.ShapeDtypeStruct((N, 16), _f32),
    )(o, bt, x0, W3, b3, W4, b4)


def kernel(x, edge_index, bn_gamma, bn_beta, W1, b1, tag1_W, tag1_b,
           W2, b2, tag2_W, tag2_b, W3, b3, W4, b4):
    row = edge_index[0].astype(jnp.int32)
    col = edge_index[1].astype(jnp.int32)
    per_tile = E // SUBCORES
    pad_per_tile = CH_PER_TILE * CHUNK - per_tile
    pad_r = jnp.broadcast_to(jnp.arange(pad_per_tile, dtype=jnp.int32) % N,
                             (SUBCORES, pad_per_tile))
    pad_c = jnp.broadcast_to(
        N + (jnp.arange(pad_per_tile, dtype=jnp.int32) % (NPAD - N)),
        (SUBCORES, pad_per_tile))
    rowp = jnp.concatenate([row.reshape(SUBCORES, per_tile), pad_r], axis=1)
    row4 = jnp.stack([2 * rowp, 2 * rowp + 1]
                     ).reshape(CORES, SUBCORES, CH_PER_TILE, CHUNK)
    col3 = jnp.concatenate([col.reshape(SUBCORES, per_tile), pad_c], axis=1
                           ).reshape(SUBCORES, CH_PER_TILE, CHUNK)
    zeros_slab = jnp.zeros((ZROWS, DH), _f32)
    zeros_n = jnp.zeros((NPAD,), _f32)

    degp = _deg(col3, zeros_n)
    x0, h, out = _pre_a(x, bn_gamma, bn_beta, W1, b1, tag1_W[0])
    g, dv = _pre_b(h, degp)
    for k in (1, 2, 3):
        p = _seg_sum(g.reshape(2 * N, DH), row4, col3, zeros_slab)
        out, g = _hop(p.reshape(NPAD, D), dv, out, tag1_W[k])
    out, g = _mid(out, tag1_b, W2, b2, tag2_W[0], dv)
    for k in (1, 2, 3):
        p = _seg_sum(g.reshape(2 * N, DH), row4, col3, zeros_slab)
        out, g = _hop(p.reshape(NPAD, D), dv, out, tag2_W[k])
    return _fin(out, tag2_b, x0, W3, b3, W4, b4)


# speedup vs baseline: 1.4407x; 1.4407x over previous
import functools

import jax
import jax.numpy as jnp
from jax import lax
from jax.experimental import pallas as pl
from jax.experimental.pallas import tpu as pltpu
from jax.experimental.pallas import tpu_sc as plsc

N = 10000
E = 320000
D = 64
D_IN = 128
CORES = 2
SUBCORES = 16
TILES = CORES * SUBCORES
CHUNK = 128
CH_PER_TILE = 80
EPAD = TILES * CH_PER_TILE * CHUNK
NPAD = 10240
ZROWS = NPAD // SUBCORES

_mesh = plsc.VectorSubcoreMesh(core_axis_name="c", subcore_axis_name="s")
_f32 = jnp.float32


def _seg_body(g_hbm, row_hbm, col_hbm, zero_hbm, out_hbm, row_v, col_v,
              rows0, rows1, gs0, gs1, acc):
    c = lax.axis_index("c")
    s = lax.axis_index("s")
    w = c * SUBCORES + s
    pltpu.sync_copy(zero_hbm, acc.at[pl.ds(s * ZROWS, ZROWS)])
    pltpu.sync_copy(row_hbm.at[w], row_v)
    pltpu.sync_copy(col_hbm.at[w], col_v)
    plsc.subcore_barrier()

    pltpu.async_copy(g_hbm.at[row_v.at[0]], rows0, gs0)

    @pl.loop(0, CH_PER_TILE, step=2)
    def _(j):
        pltpu.async_copy(g_hbm.at[row_v.at[j + 1]], rows1, gs1)
        pltpu.make_async_copy(g_hbm.at[row_v.at[j]], rows0, gs0).wait()
        pltpu.sync_copy(rows0, acc.at[col_v.at[j]], add=True)

        @pl.when(j + 2 < CH_PER_TILE)
        def _():
            pltpu.async_copy(g_hbm.at[row_v.at[j + 2]], rows0, gs0)

        pltpu.make_async_copy(g_hbm.at[row_v.at[j + 1]], rows1, gs1).wait()
        pltpu.sync_copy(rows1, acc.at[col_v.at[j + 1]], add=True)

    plsc.subcore_barrier()
    pltpu.sync_copy(acc.at[pl.ds(s * ZROWS, ZROWS)],
                    out_hbm.at[c].at[pl.ds(s * ZROWS, ZROWS)])


@jax.jit
def _seg_sum(g, row3, col3, zeros_slab):
    return pl.kernel(
        _seg_body,
        out_type=jax.ShapeDtypeStruct((CORES, NPAD, D), _f32),
        mesh=_mesh,
        scratch_types=[
            pltpu.VMEM((CH_PER_TILE, CHUNK), jnp.int32),
            pltpu.VMEM((CH_PER_TILE, CHUNK), jnp.int32),
            pltpu.VMEM((CHUNK, D), _f32),
            pltpu.VMEM((CHUNK, D), _f32),
            pltpu.SemaphoreType.DMA,
            pltpu.SemaphoreType.DMA,
            pltpu.VMEM_SHARED((NPAD, D), _f32),
        ],
        compiler_params=pltpu.CompilerParams(use_tc_tiling_on_sc=False),
    )(g, row3, col3, zeros_slab)


def _deg_body(col_hbm, zero_hbm, out_hbm, col_v, acc):
    c = lax.axis_index("c")
    s = lax.axis_index("s")
    w = c * SUBCORES + s
    pltpu.sync_copy(zero_hbm, acc)
    pltpu.sync_copy(col_hbm.at[w], col_v)
    ones = jnp.full((16,), 1.0, _f32)

    @pl.loop(0, CH_PER_TILE)
    def _(j):
        @pl.loop(0, CHUNK // 16)
        def _(i):
            idx = col_v[j, pl.ds(i * 16, 16)]
            plsc.addupdate_scatter(acc, [idx], ones)

    pltpu.sync_copy(acc, out_hbm.at[w])


@jax.jit
def _deg(col3, zeros_n):
    return pl.kernel(
        _deg_body,
        out_type=jax.ShapeDtypeStruct((TILES, NPAD), _f32),
        mesh=_mesh,
        scratch_types=[
            pltpu.VMEM((CH_PER_TILE, CHUNK), jnp.int32),
            pltpu.VMEM((NPAD,), _f32),
        ],
        compiler_params=pltpu.CompilerParams(needs_layout_passes=False),
    )(col3, zeros_n)


_DOT = functools.partial(jnp.dot, preferred_element_type=_f32,
                         precision=lax.Precision.HIGHEST)


def _pre_a_body(x_ref, gam_ref, bet_ref, W1_ref, b1_ref, tw0_ref,
                x0_ref, h_ref, out0_ref):
    x = x_ref[...]
    mean = jnp.mean(x, axis=0, keepdims=True)
    xc = x - mean
    var = jnp.mean(xc * xc, axis=0, keepdims=True)
    x0 = xc * lax.rsqrt(var + 1e-5)
    x0 = x0 * gam_ref[...][None, :] + bet_ref[...][None, :]
    x0_ref[...] = x0
    h = jnp.maximum(_DOT(x0, W1_ref[...]) + b1_ref[...][None, :], 0.0)
    h_ref[...] = h
    out0_ref[...] = _DOT(h, tw0_ref[...])


@jax.jit
def _pre_a(x, gam, bet, W1, b1, tw0):
    return pl.pallas_call(
        _pre_a_body,
        out_shape=(
            jax.ShapeDtypeStruct((N, D_IN), _f32),
            jax.ShapeDtypeStruct((N, D), _f32),
            jax.ShapeDtypeStruct((N, D), _f32),
        ),
    )(x, gam, bet, W1, b1, tw0)


def _pre_b_body(h_ref, degp_ref, g0_ref, dv_ref):
    deg = lax.dot_general(degp_ref[...], jnp.ones((TILES, 1), _f32),
                          (((0,), (0,)), ((), ())),
                          preferred_element_type=_f32,
                          precision=lax.Precision.HIGHEST)[:N]
    dinv = jnp.where(deg > 0.0, lax.rsqrt(jnp.maximum(deg, 1e-30)), 0.0)
    dv = jnp.broadcast_to(dinv, (N, D))
    dv_ref[...] = dv
    g0_ref[...] = h_ref[...] * dv


@jax.jit
def _pre_b(h, degp):
    return pl.pallas_call(
        _pre_b_body,
        out_shape=(
            jax.ShapeDtypeStruct((N, D), _f32),
            jax.ShapeDtypeStruct((N, D), _f32),
        ),
    )(h, degp)


def _hop_body(p_ref, dv_ref, acc_ref, tw_ref, out_ref, g_ref):
    dv = dv_ref[...]
    h = (p_ref[0] + p_ref[1])[:N] * dv
    out_ref[...] = acc_ref[...] + _DOT(h, tw_ref[...])
    g_ref[...] = h * dv


@jax.jit
def _hop(p, dv, acc, tw):
    return pl.pallas_call(
        _hop_body,
        out_shape=(
            jax.ShapeDtypeStruct((N, D), _f32),
            jax.ShapeDtypeStruct((N, D), _f32),
        ),
    )(p, dv, acc, tw)


def _hop_last_body(p_ref, dv_ref, acc_ref, tw_ref, out_ref):
    h = (p_ref[0] + p_ref[1])[:N] * dv_ref[...]
    out_ref[...] = acc_ref[...] + _DOT(h, tw_ref[...])


@jax.jit
def _hop_last(p, dv, acc, tw):
    return pl.pallas_call(
        _hop_last_body,
        out_shape=jax.ShapeDtypeStruct((N, D), _f32),
    )(p, dv, acc, tw)


def _mid_body(o_ref, bt_ref, W2_ref, b2_ref, tw0_ref, dv_ref,
              out0_ref, g_ref):
    h1 = jnp.maximum(o_ref[...] + bt_ref[...][None, :], 0.0)
    h2 = jnp.maximum(_DOT(h1, W2_ref[...]) + b2_ref[...][None, :], 0.0)
    out0_ref[...] = _DOT(h2, tw0_ref[...])
    g_ref[...] = h2 * dv_ref[...]


@jax.jit
def _mid(o, bt, W2, b2, tw0, dv):
    return pl.pallas_call(
        _mid_body,
        out_shape=(
            jax.ShapeDtypeStruct((N, D), _f32),
            jax.ShapeDtypeStruct((N, D), _f32),
        ),
    )(o, bt, W2, b2, tw0, dv)


def _fin_body(o_ref, bt_ref, x0_ref, W3_ref, b3_ref, W4_ref, b4_ref, y_ref):
    h = jnp.maximum(o_ref[...] + bt_ref[...][None, :], 0.0)
    W3 = W3_ref[...]
    a = _DOT(x0_ref[...], W3[:D_IN]) + _DOT(h, W3[D_IN:]) + b3_ref[...][None, :]
    a = jnp.maximum(a, 0.0)
    z = _DOT(a, W4_ref[...]) + b4_ref[...][None, :]
    z = z - jnp.max(z, axis=-1, keepdims=True)
    ez = jnp.exp(z)
    y_ref[...] = ez / jnp.sum(ez, axis=-1, keepdims=True)


@jax.jit
def _fin(o, bt, x0, W3, b3, W4, b4):
    return pl.pallas_call(
        _fin_body,
        out_shape=jax.ShapeDtypeStruct((N, 16), _f32),
    )(o, bt, x0, W3, b3, W4, b4)


def kernel(x, edge_index, bn_gamma, bn_beta, W1, b1, tag1_W, tag1_b,
           W2, b2, tag2_W, tag2_b, W3, b3, W4, b4):
    row = edge_index[0].astype(jnp.int32)
    col = edge_index[1].astype(jnp.int32)
    per_tile = E // TILES
    pad_per_tile = CH_PER_TILE * CHUNK - per_tile
    pad_r = jnp.broadcast_to(jnp.arange(pad_per_tile, dtype=jnp.int32) % N,
                             (TILES, pad_per_tile))
    pad_c = jnp.broadcast_to(
        N + (jnp.arange(pad_per_tile, dtype=jnp.int32) % (NPAD - N)),
        (TILES, pad_per_tile))
    row3 = jnp.concatenate([row.reshape(TILES, per_tile), pad_r], axis=1
                           ).reshape(TILES, CH_PER_TILE, CHUNK)
    col3 = jnp.concatenate([col.reshape(TILES, per_tile), pad_c], axis=1
                           ).reshape(TILES, CH_PER_TILE, CHUNK)
    zeros_slab = jnp.zeros((ZROWS, D), _f32)
    zeros_n = jnp.zeros((NPAD,), _f32)

    degp = _deg(col3, zeros_n)
    x0, h, out = _pre_a(x, bn_gamma, bn_beta, W1, b1, tag1_W[0])
    g, dv = _pre_b(h, degp)
    for k in (1, 2, 3):
        p = _seg_sum(g, row3, col3, zeros_slab)
        if k < 3:
            out, g = _hop(p, dv, out, tag1_W[k])
        else:
            out = _hop_last(p, dv, out, tag1_W[k])
    out, g = _mid(out, tag1_b, W2, b2, tag2_W[0], dv)
    for k in (1, 2, 3):
        p = _seg_sum(g, row3, col3, zeros_slab)
        if k < 3:
            out, g = _hop(p, dv, out, tag2_W[k])
        else:
            out = _hop_last(p, dv, out, tag2_W[k])
    return _fin(out, tag2_b, x0, W3, b3, W4, b4)


# speedup vs baseline: 1.4492x; 1.0059x over previous
import functools

import jax
import jax.numpy as jnp
from jax import lax
from jax.experimental import pallas as pl
from jax.experimental.pallas import tpu as pltpu
from jax.experimental.pallas import tpu_sc as plsc

N = 10000
E = 320000
D = 64
D_IN = 128
CORES = 2
SUBCORES = 16
TILES = CORES * SUBCORES
CHUNK = 128
CH_PER_TILE = 80
EPAD = TILES * CH_PER_TILE * CHUNK
NPAD = 10240
ZROWS = NPAD // SUBCORES

_mesh = plsc.VectorSubcoreMesh(core_axis_name="c", subcore_axis_name="s")
_f32 = jnp.float32


def _seg_body(g_hbm, row_hbm, col_hbm, zero_hbm, out_hbm, row_v, col_v,
              rows0, rows1, gs0, gs1, acc):
    c = lax.axis_index("c")
    s = lax.axis_index("s")
    w = c * SUBCORES + s
    pltpu.sync_copy(zero_hbm, acc.at[pl.ds(s * ZROWS, ZROWS)])
    pltpu.sync_copy(row_hbm.at[w], row_v)
    pltpu.sync_copy(col_hbm.at[w], col_v)
    plsc.subcore_barrier()

    pltpu.async_copy(g_hbm.at[row_v.at[0]], rows0, gs0)

    @pl.loop(0, CH_PER_TILE, step=2)
    def _(j):
        pltpu.async_copy(g_hbm.at[row_v.at[j + 1]], rows1, gs1)
        pltpu.make_async_copy(g_hbm.at[row_v.at[j]], rows0, gs0).wait()
        pltpu.sync_copy(rows0, acc.at[col_v.at[j]], add=True)

        @pl.when(j + 2 < CH_PER_TILE)
        def _():
            pltpu.async_copy(g_hbm.at[row_v.at[j + 2]], rows0, gs0)

        pltpu.make_async_copy(g_hbm.at[row_v.at[j + 1]], rows1, gs1).wait()
        pltpu.sync_copy(rows1, acc.at[col_v.at[j + 1]], add=True)

    plsc.subcore_barrier()
    pltpu.sync_copy(acc.at[pl.ds(s * ZROWS, ZROWS)],
                    out_hbm.at[c].at[pl.ds(s * ZROWS, ZROWS)])


@jax.jit
def _seg_sum(g, row3, col3, zeros_slab):
    return pl.kernel(
        _seg_body,
        out_type=jax.ShapeDtypeStruct((CORES, NPAD, D), _f32),
        mesh=_mesh,
        scratch_types=[
            pltpu.VMEM((CH_PER_TILE, CHUNK), jnp.int32),
            pltpu.VMEM((CH_PER_TILE, CHUNK), jnp.int32),
            pltpu.VMEM((CHUNK, D), _f32),
            pltpu.VMEM((CHUNK, D), _f32),
            pltpu.SemaphoreType.DMA,
            pltpu.SemaphoreType.DMA,
            pltpu.VMEM_SHARED((NPAD, D), _f32),
        ],
        compiler_params=pltpu.CompilerParams(use_tc_tiling_on_sc=False),
    )(g, row3, col3, zeros_slab)


def _deg_body(col_hbm, zero_hbm, out_hbm, col_v, acc):
    c = lax.axis_index("c")
    s = lax.axis_index("s")
    w = c * SUBCORES + s
    pltpu.sync_copy(zero_hbm, acc)
    pltpu.sync_copy(col_hbm.at[w], col_v)
    ones = jnp.full((16,), 1.0, _f32)

    @pl.loop(0, CH_PER_TILE)
    def _(j):
        @pl.loop(0, CHUNK // 16)
        def _(i):
            idx = col_v[j, pl.ds(i * 16, 16)]
            plsc.addupdate_scatter(acc, [idx], ones)

    pltpu.sync_copy(acc, out_hbm.at[w])


@jax.jit
def _deg(col3, zeros_n):
    return pl.kernel(
        _deg_body,
        out_type=jax.ShapeDtypeStruct((TILES, NPAD), _f32),
        mesh=_mesh,
        scratch_types=[
            pltpu.VMEM((CH_PER_TILE, CHUNK), jnp.int32),
            pltpu.VMEM((NPAD,), _f32),
        ],
        compiler_params=pltpu.CompilerParams(needs_layout_passes=False),
    )(col3, zeros_n)


_DOT = functools.partial(jnp.dot, preferred_element_type=_f32,
                         precision=lax.Precision.HIGHEST)


def _pre_a_body(x_ref, gam_ref, bet_ref, W1_ref, b1_ref, tw0_ref,
                x0_ref, h_ref, out0_ref):
    x = x_ref[...]
    mean = jnp.mean(x, axis=0, keepdims=True)
    xc = x - mean
    var = jnp.mean(xc * xc, axis=0, keepdims=True)
    x0 = xc * lax.rsqrt(var + 1e-5)
    x0 = x0 * gam_ref[...][None, :] + bet_ref[...][None, :]
    x0_ref[...] = x0
    h = jnp.maximum(_DOT(x0, W1_ref[...]) + b1_ref[...][None, :], 0.0)
    h_ref[...] = h
    out0_ref[...] = _DOT(h, tw0_ref[...])


@jax.jit
def _pre_a(x, gam, bet, W1, b1, tw0):
    return pl.pallas_call(
        _pre_a_body,
        out_shape=(
            jax.ShapeDtypeStruct((N, D_IN), _f32),
            jax.ShapeDtypeStruct((N, D), _f32),
            jax.ShapeDtypeStruct((N, D), _f32),
        ),
    )(x, gam, bet, W1, b1, tw0)


def _pre_b_body(h_ref, degp_ref, g0_ref, dv_ref, dv2_ref):
    deg = lax.dot_general(degp_ref[...], jnp.ones((TILES, 1), _f32),
                          (((0,), (0,)), ((), ())),
                          preferred_element_type=_f32,
                          precision=lax.Precision.HIGHEST)[:N]
    dinv = jnp.where(deg > 0.0, lax.rsqrt(jnp.maximum(deg, 1e-30)), 0.0)
    dv = jnp.broadcast_to(dinv, (N, D))
    dv_ref[...] = dv
    dv2_ref[...] = dv * dv
    g0_ref[...] = h_ref[...] * dv


@jax.jit
def _pre_b(h, degp):
    return pl.pallas_call(
        _pre_b_body,
        out_shape=(
            jax.ShapeDtypeStruct((N, D), _f32),
            jax.ShapeDtypeStruct((N, D), _f32),
            jax.ShapeDtypeStruct((N, D), _f32),
        ),
    )(h, degp)


def _hop_g_body(p_ref, dv2_ref, g_ref):
    g_ref[...] = (p_ref[0] + p_ref[1])[:N] * dv2_ref[...]


@jax.jit
def _hop_g(p, dv2):
    return pl.pallas_call(
        _hop_g_body,
        out_shape=jax.ShapeDtypeStruct((N, D), _f32),
    )(p, dv2)


def _hop_last_body(p_ref, dv_ref, acc_ref, tw_ref, out_ref):
    h = (p_ref[0] + p_ref[1])[:N] * dv_ref[...]
    out_ref[...] = acc_ref[...] + _DOT(h, tw_ref[...])


@jax.jit
def _hop_last(p, dv, acc, tw):
    return pl.pallas_call(
        _hop_last_body,
        out_shape=jax.ShapeDtypeStruct((N, D), _f32),
    )(p, dv, acc, tw)


def _mid_body(o_ref, bt_ref, W2_ref, b2_ref, tw0_ref, dv_ref,
              out0_ref, g_ref):
    h1 = jnp.maximum(o_ref[...] + bt_ref[...][None, :], 0.0)
    h2 = jnp.maximum(_DOT(h1, W2_ref[...]) + b2_ref[...][None, :], 0.0)
    out0_ref[...] = _DOT(h2, tw0_ref[...])
    g_ref[...] = h2 * dv_ref[...]


@jax.jit
def _mid(o, bt, W2, b2, tw0, dv):
    return pl.pallas_call(
        _mid_body,
        out_shape=(
            jax.ShapeDtypeStruct((N, D), _f32),
            jax.ShapeDtypeStruct((N, D), _f32),
        ),
    )(o, bt, W2, b2, tw0, dv)


def _fin_body(o_ref, bt_ref, x0_ref, W3_ref, b3_ref, W4_ref, b4_ref, y_ref):
    h = jnp.maximum(o_ref[...] + bt_ref[...][None, :], 0.0)
    W3 = W3_ref[...]
    a = _DOT(x0_ref[...], W3[:D_IN]) + _DOT(h, W3[D_IN:]) + b3_ref[...][None, :]
    a = jnp.maximum(a, 0.0)
    z = _DOT(a, W4_ref[...]) + b4_ref[...][None, :]
    z = z - jnp.max(z, axis=-1, keepdims=True)
    ez = jnp.exp(z)
    y_ref[...] = ez / jnp.sum(ez, axis=-1, keepdims=True)


@jax.jit
def _fin(o, bt, x0, W3, b3, W4, b4):
    return pl.pallas_call(
        _fin_body,
        out_shape=jax.ShapeDtypeStruct((N, 16), _f32),
    )(o, bt, x0, W3, b3, W4, b4)


def kernel(x, edge_index, bn_gamma, bn_beta, W1, b1, tag1_W, tag1_b,
           W2, b2, tag2_W, tag2_b, W3, b3, W4, b4):
    row = edge_index[0].astype(jnp.int32)
    col = edge_index[1].astype(jnp.int32)
    per_tile = E // TILES
    pad_per_tile = CH_PER_TILE * CHUNK - per_tile
    pad_r = jnp.broadcast_to(jnp.arange(pad_per_tile, dtype=jnp.int32) % N,
                             (TILES, pad_per_tile))
    pad_c = jnp.broadcast_to(
        N + (jnp.arange(pad_per_tile, dtype=jnp.int32) % (NPAD - N)),
        (TILES, pad_per_tile))
    row3 = jnp.concatenate([row.reshape(TILES, per_tile), pad_r], axis=1
                           ).reshape(TILES, CH_PER_TILE, CHUNK)
    col3 = jnp.concatenate([col.reshape(TILES, per_tile), pad_c], axis=1
                           ).reshape(TILES, CH_PER_TILE, CHUNK)
    zeros_slab = jnp.zeros((ZROWS, D), _f32)
    zeros_n = jnp.zeros((NPAD,), _f32)

    degp = _deg(col3, zeros_n)
    x0, h, out = _pre_a(x, bn_gamma, bn_beta, W1, b1, tag1_W[0])
    g, dv, dv2 = _pre_b(h, degp)
    for tw in (tag1_W, tag2_W):
        for k in (1, 2, 3):
            p = _seg_sum(g, row3, col3, zeros_slab)
            if k < 3:
                g = _hop_g(p, dv2)
                out = _hop_last(p, dv, out, tw[k])
            else:
                out = _hop_last(p, dv, out, tw[k])
        if tw is tag1_W:
            out, g = _mid(out, tag1_b, W2, b2, tag2_W[0], dv)
    return _fin(out, tag2_b, x0, W3, b3, W4, b4)
